# baseline (speedup 1.0000x reference)
import functools

import jax
import jax.numpy as jnp
from jax import lax
from jax.experimental import pallas as pl
from jax.experimental.pallas import tpu as pltpu
from jax.experimental.pallas import tpu_sc as plsc

OLD = 100000
D = 128
NC = 2
NS = 16
NW = NC * NS
LANES = 16

TOTAL = 16384
RPT = TOTAL // NW
GCH = 128
NGC = RPT // GCH
EB = 128
DUMMY = TOTAL


def _mlp_body(nv_ref, nu_ref, ut_ref, w1_ref, b1_ref, w2_ref, b2_ref, o_ref):
  nv = nv_ref[...]
  nu = nu_ref[...]
  blk = nv.shape[0]
  iota = lax.broadcasted_iota(jnp.int32, (blk, 8), 1)
  onehot = (iota == nu).astype(jnp.float32)
  ut2 = ut_ref[:, 0:2]
  w1 = w1_ref[...]
  m = jnp.dot(ut2, w1[1:3, :], preferred_element_type=jnp.float32)
  h_pre = nv * w1[0:1, :] + jnp.dot(onehot, m, preferred_element_type=jnp.float32) + b1_ref[...]
  h = 0.5 * h_pre * (1.0 + lax.erf(h_pre * 0.7071067811865476))
  o_ref[...] = jnp.dot(h, w2_ref[...], preferred_element_type=jnp.float32) + b2_ref[...]


def _mlp_rows(num_values, num_units, unit_table, W1, b1, W2, b2, n_pad):
  blk = min(n_pad, 512)
  grid = n_pad // blk
  nv = jnp.zeros((n_pad, 1), jnp.float32).at[: num_values.shape[0], 0].set(num_values)
  nu = jnp.full((n_pad, 1), 0, jnp.int32).at[: num_units.shape[0], 0].set(num_units)
  ut_pad = jnp.zeros((8, 128), jnp.float32).at[:6, :2].set(unit_table)
  w1_pad = jnp.zeros((8, W1.shape[1]), jnp.float32).at[:3, :].set(W1)
  return pl.pallas_call(
      _mlp_body,
      grid=(grid,),
      in_specs=[
          pl.BlockSpec((blk, 1), lambda i: (i, 0)),
          pl.BlockSpec((blk, 1), lambda i: (i, 0)),
          pl.BlockSpec((8, 128), lambda i: (0, 0)),
          pl.BlockSpec((8, 512), lambda i: (0, 0)),
          pl.BlockSpec((1, 512), lambda i: (0, 0)),
          pl.BlockSpec((512, 128), lambda i: (0, 0)),
          pl.BlockSpec((1, 128), lambda i: (0, 0)),
      ],
      out_specs=pl.BlockSpec((blk, 128), lambda i: (i, 0)),
      out_shape=jax.ShapeDtypeStruct((n_pad, 128), jnp.float32),
  )(nv, nu, ut_pad, w1_pad, b1.reshape(1, 512), W2, b2.reshape(1, 128))


def _sc_kernel_factory(ls, ln, n_mlp):
  mesh = plsc.VectorSubcoreMesh(core_axis_name="c", subcore_axis_name="s")
  lt = ls + ln
  iters = max(1, lt.bit_length())

  @functools.partial(
      pl.kernel,
      out_type=jax.ShapeDtypeStruct((TOTAL, D), jnp.float32),
      mesh=mesh,
      compiler_params=pltpu.CompilerParams(needs_layout_passes=False),
      scratch_types=[
          pltpu.VMEM((RPT,), jnp.int32),
          pltpu.VMEM((RPT + 1, D), jnp.float32),
          pltpu.VMEM((lt + EB,), jnp.int32),
          pltpu.VMEM((EB,), jnp.int32),
          pltpu.VMEM((EB,), jnp.int32),
          pltpu.VMEM((EB,), jnp.int32),
          pltpu.VMEM((EB,), jnp.int32),
          pltpu.VMEM((EB,), jnp.int32),
          pltpu.VMEM((EB, D), jnp.float32),
          pltpu.VMEM((EB, D), jnp.float32),
          pltpu.VMEM((EB, D), jnp.float32),
          [pltpu.SemaphoreType.DMA] * 4,
          [pltpu.SemaphoreType.DMA] * 3,
      ],
  )
  def sc_kernel(ids_hbm, list_hbm, mlp_hbm, orig_hbm, new_hbm, out_hbm,
                ids_v, rows_v, list_v, sel_s, off_s, sel_n, off_n, rank_n,
                nrows_s, nrows_n, mrows_v, gsems, psems):
    t = lax.axis_index("s") * NC + lax.axis_index("c")
    base = t * RPT

    with jax.named_scope("stage"):
      pltpu.sync_copy(ids_hbm.at[pl.ds(base, RPT)], ids_v)
      cps = []
      for j in range(NGC):
        cps.append(pltpu.async_copy(
            orig_hbm.at[ids_v.at[pl.ds(j * GCH, GCH)]],
            rows_v.at[pl.ds(j * GCH, GCH)], gsems[j]))
      pltpu.sync_copy(list_hbm, list_v)

    with jax.named_scope("search"):
      def lower_bound(lo0, hi0, key):
        def it(_, carry):
          lo, hi = carry
          mid = (lo + hi) // 2
          v = list_v[pl.ds(mid, LANES)][0]
          big = v >= key
          return jnp.where(big, lo, mid + 1), jnp.where(big, mid, hi)
        lo, _ = lax.fori_loop(
            0, iters, it, (jnp.int32(lo0), jnp.int32(hi0)))
        return lo

      lo_s = lower_bound(0, ls, base)
      hi_s = lower_bound(0, ls, base + RPT)
      lo_n = lower_bound(ls, lt, base)
      hi_n = lower_bound(ls, lt, base + RPT)

    def prep(start, lo, hi, sel_b, off_b, rank_b):
      for c in range(EB // LANES):
        e = start + c * LANES + lax.iota(jnp.int32, LANES)
        pos = list_v[pl.ds(start + c * LANES, LANES)]
        m = (e >= lo) & (e < hi)
        off = jnp.where(m, jnp.clip(pos - base, 0, RPT - 1), RPT)
        idv = plsc.load_gather(
            ids_v, [jnp.clip(pos - base, 0, RPT - 1)], mask=m)
        sel_b[pl.ds(c * LANES, LANES)] = jnp.clip(idv - OLD, 0, 9999)
        off_b[pl.ds(c * LANES, LANES)] = off
        if rank_b is not None:
          rank_b[pl.ds(c * LANES, LANES)] = jnp.clip(e - ls, 0, n_mlp - 1)

    def writeback(off_b, rows_src, add_src):
      def row_body(e, _):
        off = off_b[pl.ds(e, LANES)][0]
        for k in range(D // LANES):
          v = rows_src[e, pl.ds(k * LANES, LANES)]
          if add_src is not None:
            v = v + add_src[e, pl.ds(k * LANES, LANES)]
          rows_v[off, pl.ds(k * LANES, LANES)] = v
        return 0
      lax.fori_loop(0, EB, row_body, 0)

    with jax.named_scope("patch_prefetch"):
      cnt_s = hi_s - lo_s
      cnt_n = hi_n - lo_n
      start_s = lo_s
      start_n = lo_n
      prep(start_s, lo_s, hi_s, sel_s, off_s, None)
      cp_s = pltpu.async_copy(new_hbm.at[sel_s], nrows_s, psems[0])
      prep(start_n, lo_n, hi_n, sel_n, off_n, rank_n)
      cp_n = pltpu.async_copy(new_hbm.at[sel_n], nrows_n, psems[1])
      cp_m = pltpu.async_copy(mlp_hbm.at[rank_n], mrows_v, psems[2])

    with jax.named_scope("patch_apply"):
      for j in range(NGC):
        cps[j].wait()
      cp_s.wait()
      writeback(off_s, nrows_s, None)
      cp_n.wait()
      cp_m.wait()
      writeback(off_n, nrows_n, mrows_v)

      def extra(lo, hi, cnt, sel_b, off_b, rank_b, rows_b, add):
        def bb(k, _):
          start = lo + jnp.minimum(k * EB, jnp.maximum(0, cnt - EB))
          prep(start, lo, hi, sel_b, off_b, rank_b)
          pltpu.async_copy(new_hbm.at[sel_b], rows_b, psems[0]).wait()
          if add:
            pltpu.async_copy(mlp_hbm.at[rank_b], mrows_v, psems[1]).wait()
            writeback(off_b, rows_b, mrows_v)
          else:
            writeback(off_b, rows_b, None)
          return 0
        nb2 = (cnt + EB - 1) // EB
        lax.fori_loop(1, nb2, bb, 0)

      extra(lo_s, hi_s, cnt_s, sel_s, off_s, None, nrows_s, False)
      extra(lo_n, hi_n, cnt_n, sel_n, off_n, rank_n, nrows_n, True)

    with jax.named_scope("write_linear"):
      pltpu.sync_copy(rows_v.at[pl.ds(0, RPT)], out_hbm.at[pl.ds(base, RPT)])

  return sc_kernel


def _ceil(n, m):
  return max(m, (n + m - 1) // m * m)


def kernel(input_ids, stocks_pos, num_pos, num_values, num_units,
           orig_table, new_table, unit_table, W1, b1, W2, b2):
  ids_flat = input_ids.reshape(-1)
  n_s = stocks_pos.shape[0]
  n_n = num_pos.shape[0]
  ls, ln = _ceil(n_s, EB), _ceil(n_n, EB)
  lists = jnp.full((ls + ln + EB,), DUMMY, jnp.int32)
  lists = lists.at[:n_s].set(stocks_pos).at[ls:ls + n_n].set(num_pos)

  n_pad = _ceil(ln, 512) + 512
  mlp = _mlp_rows(num_values, num_units, unit_table, W1, b1, W2, b2, n_pad)

  sc = _sc_kernel_factory(ls, ln, n_pad)
  out = sc(ids_flat, lists, mlp, orig_table, new_table)
  return out.reshape(input_ids.shape[0], input_ids.shape[1], D)


# speedup vs baseline: 6.8818x; 6.8818x over previous
import functools

import jax
import jax.numpy as jnp
from jax import lax
from jax.experimental import pallas as pl
from jax.experimental.pallas import tpu as pltpu
from jax.experimental.pallas import tpu_sc as plsc

OLD = 100000
D = 128
NC = 2
NS = 16
NW = NC * NS
LANES = 16

TOTAL = 16384
RPT = TOTAL // NW
GCH = 128
NGC = RPT // GCH
EB = 128
DUMMY = TOTAL


def _mlp_body(nv_ref, nu_ref, ut_ref, w1_ref, b1_ref, w2_ref, b2_ref, o_ref):
  nv = nv_ref[...]
  nu = nu_ref[...]
  blk = nv.shape[0]
  iota = lax.broadcasted_iota(jnp.int32, (blk, 8), 1)
  onehot = (iota == nu).astype(jnp.float32)
  ut2 = ut_ref[:, 0:2]
  w1 = w1_ref[...]
  m = jnp.dot(ut2, w1[1:3, :], preferred_element_type=jnp.float32)
  h_pre = nv * w1[0:1, :] + jnp.dot(onehot, m, preferred_element_type=jnp.float32) + b1_ref[...]
  h = 0.5 * h_pre * (1.0 + lax.erf(h_pre * 0.7071067811865476))
  o_ref[...] = jnp.dot(h, w2_ref[...], preferred_element_type=jnp.float32) + b2_ref[...]


def _mlp_rows(num_values, num_units, unit_table, W1, b1, W2, b2, n_pad):
  blk = min(n_pad, 512)
  grid = n_pad // blk
  nv = jnp.zeros((n_pad, 1), jnp.float32).at[: num_values.shape[0], 0].set(num_values)
  nu = jnp.full((n_pad, 1), 0, jnp.int32).at[: num_units.shape[0], 0].set(num_units)
  ut_pad = jnp.zeros((8, 128), jnp.float32).at[:6, :2].set(unit_table)
  w1_pad = jnp.zeros((8, W1.shape[1]), jnp.float32).at[:3, :].set(W1)
  return pl.pallas_call(
      _mlp_body,
      grid=(grid,),
      in_specs=[
          pl.BlockSpec((blk, 1), lambda i: (i, 0)),
          pl.BlockSpec((blk, 1), lambda i: (i, 0)),
          pl.BlockSpec((8, 128), lambda i: (0, 0)),
          pl.BlockSpec((8, 512), lambda i: (0, 0)),
          pl.BlockSpec((1, 512), lambda i: (0, 0)),
          pl.BlockSpec((512, 128), lambda i: (0, 0)),
          pl.BlockSpec((1, 128), lambda i: (0, 0)),
      ],
      out_specs=pl.BlockSpec((blk, 128), lambda i: (i, 0)),
      out_shape=jax.ShapeDtypeStruct((n_pad, 128), jnp.float32),
  )(nv, nu, ut_pad, w1_pad, b1.reshape(1, 512), W2, b2.reshape(1, 128))


def _sc_kernel_factory(ls, ln, n_mlp):
  mesh = plsc.VectorSubcoreMesh(core_axis_name="c", subcore_axis_name="s")
  lt = ls + ln
  iters = max(1, lt.bit_length())

  @functools.partial(
      pl.kernel,
      out_type=jax.ShapeDtypeStruct((TOTAL, D), jnp.float32),
      mesh=mesh,
      compiler_params=pltpu.CompilerParams(needs_layout_passes=False),
      scratch_types=[
          pltpu.VMEM((RPT,), jnp.int32),
          pltpu.VMEM((RPT + 1, D), jnp.float32),
          pltpu.VMEM((lt + EB,), jnp.int32),
          pltpu.VMEM((EB,), jnp.int32),
          pltpu.VMEM((EB,), jnp.int32),
          pltpu.VMEM((EB,), jnp.int32),
          pltpu.VMEM((EB,), jnp.int32),
          pltpu.VMEM((EB,), jnp.int32),
          pltpu.VMEM((EB, D), jnp.float32),
          pltpu.VMEM((EB, D), jnp.float32),
          pltpu.VMEM((EB, D), jnp.float32),
          [pltpu.SemaphoreType.DMA] * 4,
          [pltpu.SemaphoreType.DMA] * 3,
      ],
  )
  def sc_kernel(ids_hbm, list_hbm, mlp_hbm, orig_hbm, new_hbm, out_hbm,
                ids_v, rows_v, list_v, sel_s, off_s, sel_n, off_n, rank_n,
                nrows_s, nrows_n, mrows_v, gsems, psems):
    t = lax.axis_index("s") * NC + lax.axis_index("c")
    base = t * RPT

    with jax.named_scope("stage"):
      pltpu.sync_copy(ids_hbm.at[pl.ds(base, RPT)], ids_v)
      cps = []
      for j in range(NGC):
        cps.append(pltpu.async_copy(
            orig_hbm.at[ids_v.at[pl.ds(j * GCH, GCH)]],
            rows_v.at[pl.ds(j * GCH, GCH)], gsems[j]))
      pltpu.sync_copy(list_hbm, list_v)

    with jax.named_scope("search"):
      def lower_bound(lo0, hi0, key):
        def it(_, carry):
          lo, hi = carry
          mid = (lo + hi) // 2
          v = list_v[pl.ds(mid, LANES)][0]
          big = v >= key
          return jnp.where(big, lo, mid + 1), jnp.where(big, mid, hi)
        lo, _ = lax.fori_loop(
            0, iters, it, (jnp.int32(lo0), jnp.int32(hi0)))
        return lo

      lo_s = lower_bound(0, ls, base)
      hi_s = lower_bound(0, ls, base + RPT)
      lo_n = lower_bound(ls, lt, base)
      hi_n = lower_bound(ls, lt, base + RPT)

    def prep(start, lo, hi, sel_b, off_b, rank_b):
      for c in range(EB // LANES):
        e = start + c * LANES + lax.iota(jnp.int32, LANES)
        pos = list_v[pl.ds(start + c * LANES, LANES)]
        m = (e >= lo) & (e < hi)
        off = jnp.where(m, jnp.clip(pos - base, 0, RPT - 1), RPT)
        idv = plsc.load_gather(
            ids_v, [jnp.clip(pos - base, 0, RPT - 1)], mask=m)
        filler = t * EB + c * LANES + lax.iota(jnp.int32, LANES)
        sel_b[pl.ds(c * LANES, LANES)] = jnp.where(
            m, jnp.clip(idv - OLD, 0, 9999), filler)
        off_b[pl.ds(c * LANES, LANES)] = off
        if rank_b is not None:
          rank_b[pl.ds(c * LANES, LANES)] = jnp.where(
              m, jnp.clip(e - ls, 0, n_mlp - 1), filler % 1024)

    def writeback(off_b, rows_src, add_src, nin):
      def row_body(e, _):
        off = off_b[pl.ds(e, LANES)][0]
        for k in range(D // LANES):
          v = rows_src[e, pl.ds(k * LANES, LANES)]
          if add_src is not None:
            v = v + add_src[e, pl.ds(k * LANES, LANES)]
          rows_v[off, pl.ds(k * LANES, LANES)] = v
        return 0
      lax.fori_loop(0, nin, row_body, 0)

    with jax.named_scope("patch_prefetch"):
      cnt_s = hi_s - lo_s
      cnt_n = hi_n - lo_n
      start_s = lo_s
      start_n = lo_n
      prep(start_s, lo_s, hi_s, sel_s, off_s, None)
      cp_s = pltpu.async_copy(new_hbm.at[sel_s], nrows_s, psems[0])
      prep(start_n, lo_n, hi_n, sel_n, off_n, rank_n)
      cp_n = pltpu.async_copy(new_hbm.at[sel_n], nrows_n, psems[1])
      cp_m = pltpu.async_copy(mlp_hbm.at[rank_n], mrows_v, psems[2])

    with jax.named_scope("patch_apply"):
      for j in range(NGC):
        cps[j].wait()
      cp_s.wait()
      writeback(off_s, nrows_s, None, jnp.clip(cnt_s, 0, EB))
      cp_n.wait()
      cp_m.wait()
      writeback(off_n, nrows_n, mrows_v, jnp.clip(cnt_n, 0, EB))

      def extra(lo, hi, cnt, sel_b, off_b, rank_b, rows_b, add):
        def bb(k, _):
          start = lo + jnp.minimum(k * EB, jnp.maximum(0, cnt - EB))
          prep(start, lo, hi, sel_b, off_b, rank_b)
          pltpu.async_copy(new_hbm.at[sel_b], rows_b, psems[0]).wait()
          if add:
            pltpu.async_copy(mlp_hbm.at[rank_b], mrows_v, psems[1]).wait()
            writeback(off_b, rows_b, mrows_v, EB)
          else:
            writeback(off_b, rows_b, None, EB)
          return 0
        nb2 = (cnt + EB - 1) // EB
        lax.fori_loop(1, nb2, bb, 0)

      extra(lo_s, hi_s, cnt_s, sel_s, off_s, None, nrows_s, False)
      extra(lo_n, hi_n, cnt_n, sel_n, off_n, rank_n, nrows_n, True)

    with jax.named_scope("write_linear"):
      pltpu.sync_copy(rows_v.at[pl.ds(0, RPT)], out_hbm.at[pl.ds(base, RPT)])

  return sc_kernel


def _ceil(n, m):
  return max(m, (n + m - 1) // m * m)


def kernel(input_ids, stocks_pos, num_pos, num_values, num_units,
           orig_table, new_table, unit_table, W1, b1, W2, b2):
  ids_flat = input_ids.reshape(-1)
  n_s = stocks_pos.shape[0]
  n_n = num_pos.shape[0]
  ls, ln = _ceil(n_s, EB), _ceil(n_n, EB)
  lists = jnp.full((ls + ln + EB,), DUMMY, jnp.int32)
  lists = lists.at[:n_s].set(stocks_pos).at[ls:ls + n_n].set(num_pos)

  n_pad = _ceil(ln, 512) + 512
  mlp = _mlp_rows(num_values, num_units, unit_table, W1, b1, W2, b2, n_pad)

  sc = _sc_kernel_factory(ls, ln, n_pad)
  out = sc(ids_flat, lists, mlp, orig_table, new_table)
  return out.reshape(input_ids.shape[0], input_ids.shape[1], D)


# speedup vs baseline: 7.4797x; 1.0869x over previous
import functools

import jax
import jax.numpy as jnp
from jax import lax
from jax.experimental import pallas as pl
from jax.experimental.pallas import tpu as pltpu
from jax.experimental.pallas import tpu_sc as plsc

OLD = 100000
D = 128
NC = 2
NS = 16
NW = NC * NS
LANES = 16

TOTAL = 16384
RPT = TOTAL // NW
GCH = 128
NGC = RPT // GCH
EB = 128
DUMMY = TOTAL


def _mlp_body(nv_ref, nu_ref, ut_ref, w1_ref, b1_ref, w2_ref, b2_ref, o_ref):
  nv = nv_ref[...]
  nu = nu_ref[...]
  blk = nv.shape[0]
  iota = lax.broadcasted_iota(jnp.int32, (blk, 6), 1)
  onehot = (iota == nu).astype(jnp.float32)
  w1 = w1_ref[...]
  m = jnp.dot(ut_ref[...], w1[1:3, :], preferred_element_type=jnp.float32)
  h_pre = nv * w1[0:1, :] + jnp.dot(onehot, m, preferred_element_type=jnp.float32) + b1_ref[...]
  h = 0.5 * h_pre * (1.0 + lax.erf(h_pre * 0.7071067811865476))
  o_ref[...] = jnp.dot(h, w2_ref[...], preferred_element_type=jnp.float32) + b2_ref[...]


def _mlp_rows(num_values, num_units, unit_table, W1, b1, W2, b2, n_pad):
  n = num_values.shape[0]
  if n == 0:
    num_values = jnp.zeros((1,), jnp.float32)
    num_units = jnp.zeros((1,), jnp.int32)
    n = 1
  blk = 512
  grid = (n + blk - 1) // blk
  return pl.pallas_call(
      _mlp_body,
      grid=(grid,),
      in_specs=[
          pl.BlockSpec((blk, 1), lambda i: (i, 0)),
          pl.BlockSpec((blk, 1), lambda i: (i, 0)),
          pl.BlockSpec((6, 2), lambda i: (0, 0)),
          pl.BlockSpec((3, 512), lambda i: (0, 0)),
          pl.BlockSpec((1, 512), lambda i: (0, 0)),
          pl.BlockSpec((512, 128), lambda i: (0, 0)),
          pl.BlockSpec((1, 128), lambda i: (0, 0)),
      ],
      out_specs=pl.BlockSpec((blk, 128), lambda i: (i, 0)),
      out_shape=jax.ShapeDtypeStruct((n_pad, 128), jnp.float32),
  )(num_values.reshape(-1, 1), num_units.reshape(-1, 1), unit_table, W1,
    b1.reshape(1, 512), W2, b2.reshape(1, 128))


def _sc_kernel_factory(ls, ln, n_mlp, n_s, n_n):
  mesh = plsc.VectorSubcoreMesh(core_axis_name="c", subcore_axis_name="s")
  lt = ls + ln
  iters = max(1, lt.bit_length())

  @functools.partial(
      pl.kernel,
      out_type=jax.ShapeDtypeStruct((TOTAL, D), jnp.float32),
      mesh=mesh,
      compiler_params=pltpu.CompilerParams(needs_layout_passes=False),
      scratch_types=[
          pltpu.VMEM((RPT,), jnp.int32),
          pltpu.VMEM((RPT + 1, D), jnp.float32),
          pltpu.VMEM((lt + EB,), jnp.int32),
          pltpu.VMEM((EB,), jnp.int32),
          pltpu.VMEM((EB,), jnp.int32),
          pltpu.VMEM((EB,), jnp.int32),
          pltpu.VMEM((EB,), jnp.int32),
          pltpu.VMEM((EB,), jnp.int32),
          pltpu.VMEM((EB, D), jnp.float32),
          pltpu.VMEM((EB, D), jnp.float32),
          pltpu.VMEM((EB, D), jnp.float32),
          [pltpu.SemaphoreType.DMA] * 4,
          [pltpu.SemaphoreType.DMA] * 3,
      ],
  )
  def sc_kernel(ids_hbm, sp_hbm, np_hbm, mlp_hbm, orig_hbm, new_hbm, out_hbm,
                ids_v, rows_v, list_v, sel_s, off_s, sel_n, off_n, rank_n,
                nrows_s, nrows_n, mrows_v, gsems, psems):
    t = lax.axis_index("s") * NC + lax.axis_index("c")
    base = t * RPT

    with jax.named_scope("stage"):
      pltpu.sync_copy(ids_hbm.at[pl.ds(base, RPT)], ids_v)
      cps = []
      for j in range(NGC):
        cps.append(pltpu.async_copy(
            orig_hbm.at[ids_v.at[pl.ds(j * GCH, GCH)]],
            rows_v.at[pl.ds(j * GCH, GCH)], gsems[j]))
      dvec = jnp.full((LANES,), DUMMY, jnp.int32)
      def fill(z, _):
        list_v[pl.ds(z * LANES, LANES)] = dvec
        return 0
      lax.fori_loop(0, (lt + EB) // LANES, fill, 0)
      if n_s > 0:
        pltpu.sync_copy(sp_hbm, list_v.at[pl.ds(0, n_s)])
      if n_n > 0:
        pltpu.sync_copy(np_hbm, list_v.at[pl.ds(ls, n_n)])

    with jax.named_scope("search"):
      def lower_bound(lo0, hi0, key):
        def it(_, carry):
          lo, hi = carry
          mid = (lo + hi) // 2
          v = list_v[pl.ds(mid, LANES)][0]
          big = v >= key
          return jnp.where(big, lo, mid + 1), jnp.where(big, mid, hi)
        lo, _ = lax.fori_loop(
            0, iters, it, (jnp.int32(lo0), jnp.int32(hi0)))
        return lo

      lo_s = lower_bound(0, ls, base)
      hi_s = lower_bound(0, ls, base + RPT)
      lo_n = lower_bound(ls, lt, base)
      hi_n = lower_bound(ls, lt, base + RPT)

    def prep(start, lo, hi, sel_b, off_b, rank_b):
      for c in range(EB // LANES):
        e = start + c * LANES + lax.iota(jnp.int32, LANES)
        pos = list_v[pl.ds(start + c * LANES, LANES)]
        m = (e >= lo) & (e < hi)
        off = jnp.where(m, jnp.clip(pos - base, 0, RPT - 1), RPT)
        idv = plsc.load_gather(
            ids_v, [jnp.clip(pos - base, 0, RPT - 1)], mask=m)
        filler = t * EB + c * LANES + lax.iota(jnp.int32, LANES)
        sel_b[pl.ds(c * LANES, LANES)] = jnp.where(
            m, jnp.clip(idv - OLD, 0, 9999), filler)
        off_b[pl.ds(c * LANES, LANES)] = off
        if rank_b is not None:
          rank_b[pl.ds(c * LANES, LANES)] = jnp.where(
              m, jnp.clip(e - ls, 0, n_mlp - 1), filler % 1024)

    def writeback(off_b, rows_src, add_src, nin):
      def row_body(e, _):
        off = off_b[pl.ds(e, LANES)][0]
        for k in range(D // LANES):
          v = rows_src[e, pl.ds(k * LANES, LANES)]
          if add_src is not None:
            v = v + add_src[e, pl.ds(k * LANES, LANES)]
          rows_v[off, pl.ds(k * LANES, LANES)] = v
        return 0
      lax.fori_loop(0, nin, row_body, 0)

    with jax.named_scope("patch_prefetch"):
      cnt_s = hi_s - lo_s
      cnt_n = hi_n - lo_n
      start_s = lo_s
      start_n = lo_n
      prep(start_s, lo_s, hi_s, sel_s, off_s, None)
      cp_s = pltpu.async_copy(new_hbm.at[sel_s], nrows_s, psems[0])
      prep(start_n, lo_n, hi_n, sel_n, off_n, rank_n)
      cp_n = pltpu.async_copy(new_hbm.at[sel_n], nrows_n, psems[1])
      cp_m = pltpu.async_copy(mlp_hbm.at[rank_n], mrows_v, psems[2])

    with jax.named_scope("patch_apply"):
      for j in range(NGC):
        cps[j].wait()
      cp_s.wait()
      writeback(off_s, nrows_s, None, jnp.clip(cnt_s, 0, EB))
      cp_n.wait()
      cp_m.wait()
      writeback(off_n, nrows_n, mrows_v, jnp.clip(cnt_n, 0, EB))

      def extra(lo, hi, cnt, sel_b, off_b, rank_b, rows_b, add):
        def bb(k, _):
          start = lo + jnp.minimum(k * EB, jnp.maximum(0, cnt - EB))
          prep(start, lo, hi, sel_b, off_b, rank_b)
          pltpu.async_copy(new_hbm.at[sel_b], rows_b, psems[0]).wait()
          if add:
            pltpu.async_copy(mlp_hbm.at[rank_b], mrows_v, psems[1]).wait()
            writeback(off_b, rows_b, mrows_v, EB)
          else:
            writeback(off_b, rows_b, None, EB)
          return 0
        nb2 = (cnt + EB - 1) // EB
        lax.fori_loop(1, nb2, bb, 0)

      extra(lo_s, hi_s, cnt_s, sel_s, off_s, None, nrows_s, False)
      extra(lo_n, hi_n, cnt_n, sel_n, off_n, rank_n, nrows_n, True)

    with jax.named_scope("write_linear"):
      pltpu.sync_copy(rows_v.at[pl.ds(0, RPT)], out_hbm.at[pl.ds(base, RPT)])

  return sc_kernel


def _ceil(n, m):
  return max(m, (n + m - 1) // m * m)


def kernel(input_ids, stocks_pos, num_pos, num_values, num_units,
           orig_table, new_table, unit_table, W1, b1, W2, b2):
  ids_flat = input_ids.reshape(-1)
  n_s = stocks_pos.shape[0]
  n_n = num_pos.shape[0]
  ls, ln = _ceil(n_s, EB), _ceil(n_n, EB)
  if n_s == 0:
    stocks_pos = jnp.zeros((1,), jnp.int32)
  if n_n == 0:
    num_pos = jnp.zeros((1,), jnp.int32)

  n_pad = _ceil(ln, 512) + 512
  mlp = _mlp_rows(num_values, num_units, unit_table, W1, b1, W2, b2, n_pad)

  sc = _sc_kernel_factory(ls, ln, n_pad, n_s, n_n)
  out = sc(ids_flat, stocks_pos, num_pos, mlp, orig_table, new_table)
  return out.reshape(input_ids.shape[0], input_ids.shape[1], D)
